# Initial kernel scaffold; baseline (speedup 1.0000x reference)
import jax
import jax.numpy as jnp
from jax.experimental import pallas as pl


def kernel(seq, edge_index, subgraphs, subgraph_logits_ids, W1a, b1a, W1b, b1b, W2a, b2a, W2b, b2b):
    raise NotImplementedError("write your pallas kernel here")


# speedup vs baseline: 2.3299x; 2.3299x over previous
import functools

import jax
import jax.numpy as jnp
from jax import lax
from jax.experimental import pallas as pl
from jax.experimental.pallas import tpu as pltpu
from jax.experimental.pallas import tpu_sc as plsc

N = 10000
E = 320000
D = 128
OUT_D = 256
NS = 500
SAMP = 10
SUB = 20
TEMPERATURE = 10.0

NC = 2
NSUB = 16
NW = NC * NSUB

EC = 128
CPT = 80
E_PAD = NW * CPT * EC
N_ACC = 10240
RPT = N_ACC // NSUB

UNITS = NS * (SAMP + 2)
UPT = 192
U_PAD = NW * UPT

_mesh = plsc.VectorSubcoreMesh(core_axis_name="c", subcore_axis_name="s")


def _segsum_body(h_hbm, src_hbm, dst_hbm, out_hbm, srcv, dstv, rows_v, zv, acc_sh, sem):
    c = lax.axis_index("c")
    s = lax.axis_index("s")
    w = s * NC + c

    zf = jnp.zeros((16,), jnp.float32)
    for r in range(16):
        for g in range(D // 16):
            zv[r, pl.ds(g * 16, 16)] = zf
    base_row = s * RPT

    def zloop(t, carry):
        pltpu.sync_copy(zv, acc_sh.at[pl.ds(base_row + t * 16, 16)])
        return carry

    lax.fori_loop(0, RPT // 16, zloop, 0)
    plsc.subcore_barrier()

    crow = w * CPT
    pltpu.sync_copy(src_hbm.at[pl.ds(crow, CPT)], srcv)
    pltpu.sync_copy(dst_hbm.at[pl.ds(crow, CPT)], dstv)

    def eloop(j, carry):
        pltpu.async_copy(h_hbm.at[srcv.at[j]], rows_v, sem).wait()
        pltpu.sync_copy(rows_v, acc_sh.at[dstv.at[j]], add=True)
        return carry

    lax.fori_loop(0, CPT, eloop, 0)
    plsc.subcore_barrier()

    pltpu.sync_copy(acc_sh.at[pl.ds(base_row, RPT)],
                    out_hbm.at[pl.ds(c * N_ACC + base_row, RPT)])


_segsum = functools.partial(
    pl.kernel,
    mesh=_mesh,
    out_type=jax.ShapeDtypeStruct((NC * N_ACC, D), jnp.float32),
    scratch_types=[
        pltpu.VMEM((CPT, EC), jnp.int32),
        pltpu.VMEM((CPT, EC), jnp.int32),
        pltpu.VMEM((EC, D), jnp.float32),
        pltpu.VMEM((16, D), jnp.float32),
        pltpu.VMEM_SHARED((N_ACC, D), jnp.float32),
        pltpu.SemaphoreType.DMA,
    ],
)(_segsum_body)


def _readout_body(logits_hbm, ids_hbm, out_hbm, idsv, grows, outv, sem):
    c = lax.axis_index("c")
    s = lax.axis_index("s")
    w = s * NC + c
    base = w * UPT
    pltpu.sync_copy(ids_hbm.at[pl.ds(base, UPT)], idsv)

    def uloop(u, carry):
        pltpu.async_copy(logits_hbm.at[idsv.at[u]], grows, sem).wait()
        for g in range(OUT_D // 16):
            acc = jnp.zeros((16,), jnp.float32)
            for r in range(SUB):
                acc = acc + grows[r, pl.ds(g * 16, 16)]
            outv[u, pl.ds(g * 16, 16)] = acc * (1.0 / SUB)
        return carry

    lax.fori_loop(0, UPT, uloop, 0)
    pltpu.sync_copy(outv, out_hbm.at[pl.ds(base, UPT)])


_readout = functools.partial(
    pl.kernel,
    mesh=_mesh,
    out_type=jax.ShapeDtypeStruct((U_PAD, OUT_D), jnp.float32),
    scratch_types=[
        pltpu.VMEM((UPT, SUB), jnp.int32),
        pltpu.VMEM((SUB, OUT_D), jnp.float32),
        pltpu.VMEM((UPT, OUT_D), jnp.float32),
        pltpu.SemaphoreType.DMA,
    ],
)(_readout_body)


MB = 1000


def _mlp_body(concat_in, h_ref, a0_ref, a1_ref, wa_ref, ba_ref, wb_ref, bb_ref, out_ref):
    hin = h_ref[...]
    m = hin + a0_ref[...] + a1_ref[...]
    t = jnp.maximum(jnp.dot(m, wa_ref[...], preferred_element_type=jnp.float32)
                    + ba_ref[...], 0.0)
    h2 = jnp.maximum(jnp.dot(t, wb_ref[...], preferred_element_type=jnp.float32)
                     + bb_ref[...], 0.0)
    if concat_in:
        out_ref[:, :D] = hin
        out_ref[:, D:] = h2
    else:
        out_ref[...] = h2


def _mlp(h, parts, Wa, ba, Wb, bb, concat_in):
    out_d = OUT_D if concat_in else D
    return pl.pallas_call(
        functools.partial(_mlp_body, concat_in),
        grid=(N // MB,),
        in_specs=[
            pl.BlockSpec((MB, D), lambda i: (i, 0)),
            pl.BlockSpec((MB, D), lambda i: (i, 0)),
            pl.BlockSpec((MB, D), lambda i: (i, 0)),
            pl.BlockSpec((D, D), lambda i: (0, 0)),
            pl.BlockSpec((1, D), lambda i: (0, 0)),
            pl.BlockSpec((D, D), lambda i: (0, 0)),
            pl.BlockSpec((1, D), lambda i: (0, 0)),
        ],
        out_specs=pl.BlockSpec((MB, out_d), lambda i: (i, 0)),
        out_shape=jax.ShapeDtypeStruct((N, out_d), jnp.float32),
    )(h, parts[:N_ACC], parts[N_ACC:], Wa, ba.reshape(1, D), Wb, bb.reshape(1, D))


def _loss_body(y_ref, o_ref):
    self_v = y_ref[:, 0, :]
    n2s = jnp.sum(self_v * self_v, axis=1, keepdims=True)
    nas = jnp.maximum(jnp.sqrt(n2s), 1e-8)
    es = []
    for k in range(1, SAMP + 2):
        sk = y_ref[:, k, :]
        dk = jnp.sum(self_v * sk, axis=1, keepdims=True)
        nbk = jnp.maximum(jnp.sqrt(jnp.sum(sk * sk, axis=1, keepdims=True)), 1e-8)
        sim = dk / (nas * nbk)
        es.append(jnp.exp(sim) / TEMPERATURE)
    num = es[0]
    den = es[1]
    for k in range(2, SAMP + 1):
        den = den + es[k]
    res = -jnp.log(num / den)
    o_ref[...] = (jnp.sum(res) / NS)[None, None]


def _loss(y):
    return pl.pallas_call(
        _loss_body,
        out_shape=jax.ShapeDtypeStruct((1, 1), jnp.float32),
    )(y)


def kernel(seq, edge_index, subgraphs, subgraph_logits_ids,
           W1a, b1a, W1b, b1b, W2a, b2a, W2b, b2b):
    src = edge_index[0]
    dst = edge_index[1]
    pad = E_PAD - E
    src2d = jnp.concatenate([src, jnp.zeros((pad,), jnp.int32)]).reshape(-1, EC)
    dst2d = jnp.concatenate([dst, jnp.full((pad,), N, jnp.int32)]).reshape(-1, EC)

    parts1 = _segsum(seq, src2d, dst2d)
    h1 = _mlp(seq, parts1, W1a, b1a, W1b, b1b, concat_in=False)
    parts2 = _segsum(h1, src2d, dst2d)
    logits = _mlp(h1, parts2, W2a, b2a, W2b, b2b, concat_in=True)

    ids = jnp.concatenate([subgraphs[:, None, :], subgraph_logits_ids], axis=1)
    ids_flat = ids.reshape(-1, SUB)
    ids_pad = jnp.concatenate([ids_flat, jnp.zeros((U_PAD - UNITS, SUB), jnp.int32)])

    means = _readout(logits, ids_pad)
    y = means[:UNITS].reshape(NS, SAMP + 2, OUT_D)
    return _loss(y).reshape(())
